# baseline (speedup 1.0000x reference)
import jax
import jax.numpy as jnp
from jax.experimental import pallas as pl
from jax.experimental.pallas import tpu as pltpu

_MARGIN = 0.5
_ALPHA = 0.01
_BATCHSZ = 1024.0
_K = 3
_G = 4
_C = 28


def _extract_smallest(v, iota, n):
    outs = []
    for i in range(n):
        m = jnp.min(v, axis=0, keepdims=True)
        outs.append(m)
        if i < n - 1:
            amin = jnp.min(jnp.where(v == m, iota, jnp.int32(1 << 30)),
                           axis=0, keepdims=True)
            v = jnp.where(iota == amin, jnp.inf, v)
    return outs


def _merge_chunks(rows_112):
    parts = []
    for r in rows_112:
        for g in range(_G):
            parts.append(r[:, g * _C:(g + 1) * _C])
    return jnp.concatenate(parts, axis=0)


def _tile_lanes(x):
    return jnp.concatenate([x] * _G, axis=1)


def _crl_kernel(xt_ref, out_ref):
    xp = xt_ref[0:256, :]
    tp = xt_ref[256:512, :]
    Rp, W = xp.shape
    R = Rp * _G

    sig = jax.nn.sigmoid(xp)

    nonneg = xp >= 0.0
    bce = jnp.sum(jnp.maximum(xp, 0.0) - xp * tp
                  - jnp.log(jnp.where(nonneg, sig, 1.0 - sig))) / (R * _C)

    per_lane_col = jax.lax.dot_general(
        tp, jnp.ones((Rp, 1), jnp.float32), (((0,), (0,)), ((), ())),
        preferred_element_type=jnp.float32)
    et = (jax.lax.broadcasted_iota(jnp.int32, (_C, W), 1) % _C
          == jax.lax.broadcasted_iota(jnp.int32, (_C, W), 0)
          ).astype(jnp.float32)
    counts_col = jax.lax.dot_general(
        et, per_lane_col, (((1,), (0,)), ((), ())),
        preferred_element_type=jnp.float32)
    per_lane_row = jnp.sum(tp, axis=0, keepdims=True)
    e = (jax.lax.broadcasted_iota(jnp.int32, (W, _C), 0) % _C
         == jax.lax.broadcasted_iota(jnp.int32, (W, _C), 1)
         ).astype(jnp.float32)
    counts_row = jax.lax.dot_general(
        per_lane_row, e, (((1,), (0,)), ((), ())),
        preferred_element_type=jnp.float32)

    k_idx = jax.lax.broadcasted_iota(jnp.int32, (_C, _C), 0)
    j_idx = jax.lax.broadcasted_iota(jnp.int32, (_C, _C), 1)
    le = (counts_col < counts_row) | (
        (counts_col == counts_row) & (k_idx <= j_idx))
    prefix = jnp.sum(jnp.where(le, counts_col, 0.0), axis=0,
                     keepdims=True)
    sel = (prefix <= 0.5 * _BATCHSZ) & (counts_row > 1.0)

    n_p = jnp.minimum(jnp.float32(_K), counts_row - 1.0)
    n_n = jnp.minimum(jnp.float32(_K), R - counts_row)

    pos = tp == 1.0
    riota = jax.lax.broadcasted_iota(jnp.int32, (Rp, W), 0)
    miota_p = jax.lax.broadcasted_iota(jnp.int32, ((_K + 1) * _G, _C), 0)
    miota_n = jax.lax.broadcasted_iota(jnp.int32, (_K * _G, _C), 0)

    s_chunk = _extract_smallest(jnp.where(pos, sig, jnp.inf), riota, _K + 1)
    s_vals = _extract_smallest(_merge_chunks(s_chunk), miota_p, _K + 1)

    u_chunk = _extract_smallest(jnp.where(pos, jnp.inf, sig), riota, _K)
    u_vals = _extract_smallest(_merge_chunks(u_chunk), miota_n, _K)

    t_f = jnp.clip(n_p, 0.0, jnp.float32(_K))
    s_t = jnp.where(t_f == 0.0, s_vals[0],
                    jnp.where(t_f == 1.0, s_vals[1],
                              jnp.where(t_f == 2.0, s_vals[2], s_vals[3])))

    s_t_w = _tile_lanes(s_t)
    n_p_w = _tile_lanes(n_p)
    n_n_w = _tile_lanes(n_n)

    L = n_p_w + (sig <= s_t_w).astype(jnp.float32)
    sum_pos = jnp.zeros_like(sig)
    for i in range(_K + 1):
        sum_pos = sum_pos + jnp.where(
            jnp.float32(i) < L, jnp.abs(sig - _tile_lanes(s_vals[i])), 0.0)
    sum_neg = jnp.zeros_like(sig)
    for i in range(_K):
        sum_neg = sum_neg + jnp.where(
            jnp.float32(i) < n_n_w, jnp.abs(sig - _tile_lanes(u_vals[i])),
            0.0)

    gate = (sel & (n_p > 0.0) & (n_n > 0.0)).astype(jnp.float32)
    vf = pos.astype(jnp.float32) * _tile_lanes(gate)
    d = jnp.sum(vf * (n_n_w * sum_pos - n_p_w * sum_neg))
    has_any = jnp.max(vf) > 0.0

    crl = jnp.maximum(d + _MARGIN, 0.0)
    out_ref[0, 0] = jnp.where(has_any, _ALPHA * crl + (1.0 - _ALPHA) * bce,
                              bce)


def _pack(a):
    return a.reshape(256, _G * _C)


def kernel(input, target, X):
    del X
    out = pl.pallas_call(
        _crl_kernel,
        out_shape=jax.ShapeDtypeStruct((1, 1), jnp.float32),
        out_specs=pl.BlockSpec(memory_space=pltpu.SMEM),
    )(jnp.concatenate([_pack(input), _pack(target)], axis=0))
    return jnp.reshape(out, ())


# speedup vs baseline: 1.0056x; 1.0056x over previous
import jax
import jax.numpy as jnp
from jax.experimental import pallas as pl
from jax.experimental.pallas import tpu as pltpu

_MARGIN = 0.5
_ALPHA = 0.01
_BATCHSZ = 1024.0
_K = 3
_G = 4
_C = 28


def _extract_smallest(v, iota, n):
    outs = []
    for i in range(n):
        m = jnp.min(v, axis=0, keepdims=True)
        outs.append(m)
        if i < n - 1:
            amin = jnp.min(jnp.where(v == m, iota, jnp.int32(1 << 30)),
                           axis=0, keepdims=True)
            v = jnp.where(iota == amin, jnp.inf, v)
    return outs


def _merge_chunks(rows_112):
    parts = []
    for r in rows_112:
        for g in range(_G):
            parts.append(r[:, g * _C:(g + 1) * _C])
    return jnp.concatenate(parts, axis=0)


def _tile_lanes(x):
    return jnp.concatenate([x] * _G, axis=1)


def _crl_kernel(xp_ref, tp_ref, out_ref):
    xp = xp_ref[:, :]
    tp = tp_ref[:, :]
    Rp, W = xp.shape
    R = Rp * _G

    sig = jax.nn.sigmoid(xp)

    nonneg = xp >= 0.0
    bce = jnp.sum(jnp.maximum(xp, 0.0) - xp * tp
                  - jnp.log(jnp.where(nonneg, sig, 1.0 - sig))) / (R * _C)

    per_lane_col = jax.lax.dot_general(
        tp, jnp.ones((Rp, 1), jnp.float32), (((0,), (0,)), ((), ())),
        preferred_element_type=jnp.float32)
    et = (jax.lax.broadcasted_iota(jnp.int32, (_C, W), 1) % _C
          == jax.lax.broadcasted_iota(jnp.int32, (_C, W), 0)
          ).astype(jnp.float32)
    counts_col = jax.lax.dot_general(
        et, per_lane_col, (((1,), (0,)), ((), ())),
        preferred_element_type=jnp.float32)
    per_lane_row = jnp.sum(tp, axis=0, keepdims=True)
    e = (jax.lax.broadcasted_iota(jnp.int32, (W, _C), 0) % _C
         == jax.lax.broadcasted_iota(jnp.int32, (W, _C), 1)
         ).astype(jnp.float32)
    counts_row = jax.lax.dot_general(
        per_lane_row, e, (((1,), (0,)), ((), ())),
        preferred_element_type=jnp.float32)

    k_idx = jax.lax.broadcasted_iota(jnp.int32, (_C, _C), 0)
    j_idx = jax.lax.broadcasted_iota(jnp.int32, (_C, _C), 1)
    le = (counts_col < counts_row) | (
        (counts_col == counts_row) & (k_idx <= j_idx))
    prefix = jnp.sum(jnp.where(le, counts_col, 0.0), axis=0,
                     keepdims=True)
    sel = (prefix <= 0.5 * _BATCHSZ) & (counts_row > 1.0)

    n_p = jnp.minimum(jnp.float32(_K), counts_row - 1.0)
    n_n = jnp.minimum(jnp.float32(_K), R - counts_row)

    pos = tp == 1.0
    riota = jax.lax.broadcasted_iota(jnp.int32, (Rp, W), 0)
    miota_p = jax.lax.broadcasted_iota(jnp.int32, ((_K + 1) * _G, _C), 0)
    miota_n = jax.lax.broadcasted_iota(jnp.int32, (_K * _G, _C), 0)

    s_chunk = _extract_smallest(jnp.where(pos, sig, jnp.inf), riota, _K + 1)
    s_vals = _extract_smallest(_merge_chunks(s_chunk), miota_p, _K + 1)

    u_chunk = _extract_smallest(jnp.where(pos, jnp.inf, sig), riota, _K)
    u_vals = _extract_smallest(_merge_chunks(u_chunk), miota_n, _K)

    t_f = jnp.clip(n_p, 0.0, jnp.float32(_K))
    s_t = jnp.where(t_f == 0.0, s_vals[0],
                    jnp.where(t_f == 1.0, s_vals[1],
                              jnp.where(t_f == 2.0, s_vals[2], s_vals[3])))

    s_t_w = _tile_lanes(s_t)
    n_p_w = _tile_lanes(n_p)
    n_n_w = _tile_lanes(n_n)

    L = n_p_w + (sig <= s_t_w).astype(jnp.float32)
    sum_pos = jnp.zeros_like(sig)
    for i in range(_K + 1):
        sum_pos = sum_pos + jnp.where(
            jnp.float32(i) < L, jnp.abs(sig - _tile_lanes(s_vals[i])), 0.0)
    sum_neg = jnp.zeros_like(sig)
    for i in range(_K):
        sum_neg = sum_neg + jnp.where(
            jnp.float32(i) < n_n_w, jnp.abs(sig - _tile_lanes(u_vals[i])),
            0.0)

    gate = (sel & (n_p > 0.0) & (n_n > 0.0)).astype(jnp.float32)
    vf = pos.astype(jnp.float32) * _tile_lanes(gate)
    d = jnp.sum(vf * (n_n_w * sum_pos - n_p_w * sum_neg))
    has_any = jnp.max(vf) > 0.0

    crl = jnp.maximum(d + _MARGIN, 0.0)
    out_ref[0, 0] = jnp.where(has_any, _ALPHA * crl + (1.0 - _ALPHA) * bce,
                              bce)


def _pack(a):
    return a.reshape(256, _G * _C)


def kernel(input, target, X):
    del X
    out = pl.pallas_call(
        _crl_kernel,
        out_shape=jax.ShapeDtypeStruct((1, 1), jnp.float32),
        out_specs=pl.BlockSpec(memory_space=pltpu.SMEM),
    )(_pack(input), _pack(target))
    return jnp.reshape(out, ())
